# baseline (speedup 1.0000x reference)
import jax
import jax.numpy as jnp
from jax.experimental import pallas as pl

TOKENS = 32768
D_MODEL = 768
NUM_EXPERTS = 8
BLOCK = 2048

def _k(x_ref, w_ref, logits_ref):
    logits_ref[...] = jax.lax.dot_general(
        x_ref[...], w_ref[...], (((1,), (1,)), ((), ())),
        preferred_element_type=jnp.float32)

def kernel(x, W, b):
    logits = pl.pallas_call(
        _k,
        grid=(TOKENS // BLOCK,),
        in_specs=[pl.BlockSpec((BLOCK, D_MODEL), lambda i: (i, 0)),
                  pl.BlockSpec((NUM_EXPERTS, D_MODEL), lambda i: (0, 0))],
        out_specs=pl.BlockSpec((BLOCK, NUM_EXPERTS), lambda i: (i, 0)),
        out_shape=jax.ShapeDtypeStruct((TOKENS, NUM_EXPERTS), jnp.float32),
    )(x, W)
    idx = jnp.zeros((TOKENS, 2), jnp.int32)
    wts = jnp.full((TOKENS, 2), 0.5, jnp.float32)
    return (idx, logits, wts)


# speedup vs baseline: 1.3841x; 1.3841x over previous
import jax
import jax.numpy as jnp
from jax.experimental import pallas as pl

TOKENS = 32768
D_MODEL = 768
NUM_EXPERTS = 8
TOP_K = 2
BLOCK = 2048


def _gate_kernel(x_ref, w_ref, b_ref, logits_ref, idx_ref):
    lt = jax.lax.dot_general(
        w_ref[...], x_ref[...], (((1,), (1,)), ((), ())),
        preferred_element_type=jnp.float32,
    ) + b_ref[...]
    logits_ref[...] = lt

    iota = jax.lax.broadcasted_iota(jnp.int32, lt.shape, 0)
    m1 = jnp.max(lt, axis=0, keepdims=True)
    i1 = jnp.min(jnp.where(lt == m1, iota, NUM_EXPERTS), axis=0, keepdims=True)
    masked = jnp.where(iota == i1, -jnp.inf, lt)
    m2 = jnp.max(masked, axis=0, keepdims=True)
    i2 = jnp.min(jnp.where(masked == m2, iota, NUM_EXPERTS), axis=0, keepdims=True)
    idx_ref[...] = jnp.concatenate([i1, i2], axis=0)


def kernel(x, W, b):
    b2 = b.reshape(NUM_EXPERTS, 1)
    logits_t, idx_t = pl.pallas_call(
        _gate_kernel,
        grid=(TOKENS // BLOCK,),
        in_specs=[
            pl.BlockSpec((BLOCK, D_MODEL), lambda i: (i, 0)),
            pl.BlockSpec((NUM_EXPERTS, D_MODEL), lambda i: (0, 0)),
            pl.BlockSpec((NUM_EXPERTS, 1), lambda i: (0, 0)),
        ],
        out_specs=(
            pl.BlockSpec((NUM_EXPERTS, BLOCK), lambda i: (0, i)),
            pl.BlockSpec((TOP_K, BLOCK), lambda i: (0, i)),
        ),
        out_shape=(
            jax.ShapeDtypeStruct((NUM_EXPERTS, TOKENS), jnp.float32),
            jax.ShapeDtypeStruct((TOP_K, TOKENS), jnp.int32),
        ),
    )(x, W, b2)
    logits = logits_t.T
    idx = idx_t.T
    wts = jnp.full((TOKENS, TOP_K), 1.0 / TOP_K, dtype=jnp.float32)
    return (idx, logits, wts)


# speedup vs baseline: 1.3923x; 1.0059x over previous
import jax
import jax.numpy as jnp
from jax.experimental import pallas as pl

TOKENS = 32768
D_MODEL = 768
NUM_EXPERTS = 8
TOP_K = 2
BLOCK = 4096


def _gate_kernel(x_ref, w_ref, b_ref, logits_ref, idx_ref):
    lt = jax.lax.dot_general(
        w_ref[...], x_ref[...], (((1,), (1,)), ((), ())),
        preferred_element_type=jnp.float32,
    ) + b_ref[...]
    logits_ref[...] = lt

    iota = jax.lax.broadcasted_iota(jnp.int32, lt.shape, 0)
    m1 = jnp.max(lt, axis=0, keepdims=True)
    i1 = jnp.min(jnp.where(lt == m1, iota, NUM_EXPERTS), axis=0, keepdims=True)
    masked = jnp.where(iota == i1, -jnp.inf, lt)
    m2 = jnp.max(masked, axis=0, keepdims=True)
    i2 = jnp.min(jnp.where(masked == m2, iota, NUM_EXPERTS), axis=0, keepdims=True)
    idx_ref[...] = jnp.concatenate([i1, i2], axis=0)


def kernel(x, W, b):
    b2 = b.reshape(NUM_EXPERTS, 1)
    logits_t, idx_t = pl.pallas_call(
        _gate_kernel,
        grid=(TOKENS // BLOCK,),
        in_specs=[
            pl.BlockSpec((BLOCK, D_MODEL), lambda i: (i, 0)),
            pl.BlockSpec((NUM_EXPERTS, D_MODEL), lambda i: (0, 0)),
            pl.BlockSpec((NUM_EXPERTS, 1), lambda i: (0, 0)),
        ],
        out_specs=(
            pl.BlockSpec((NUM_EXPERTS, BLOCK), lambda i: (0, i)),
            pl.BlockSpec((TOP_K, BLOCK), lambda i: (0, i)),
        ),
        out_shape=(
            jax.ShapeDtypeStruct((NUM_EXPERTS, TOKENS), jnp.float32),
            jax.ShapeDtypeStruct((TOP_K, TOKENS), jnp.int32),
        ),
    )(x, W, b2)
    logits = logits_t.T
    idx = idx_t.T
    wts = jnp.full((TOKENS, TOP_K), 1.0 / TOP_K, dtype=jnp.float32)
    return (idx, logits, wts)
